# baseline (speedup 1.0000x reference)
import functools
import math

import jax
import jax.numpy as jnp
from jax import lax
from jax.experimental import pallas as pl
from jax.experimental.pallas import tpu as pltpu
from jax.experimental.pallas import tpu_sc as plsc

SC_NC = 2
SC_NS = 16
SC_L = 16
SC_NW = SC_NC * SC_NS

NUM_HEADS = 16
KV_HEADS = 4
NUM_EXPERTS = 8
CAP_FACTOR = 1.25
NEG = -1e30


def _qkv_kernel(x_ref, rms1_ref, wq_ref, wk_ref, wv_ref, q_ref, k_ref, v_ref,
                *, H, Hkv, Dh):
    x = x_ref[...]
    ms = jnp.mean(x * x, axis=-1, keepdims=True)
    h = x * lax.rsqrt(ms + 1e-6) * rms1_ref[...]
    q = jnp.dot(h, wq_ref[...], preferred_element_type=jnp.float32)
    k = jnp.dot(h, wk_ref[...], preferred_element_type=jnp.float32)
    v = jnp.dot(h, wv_ref[...], preferred_element_type=jnp.float32)
    for i in range(H):
        q_ref[i] = q[:, i * Dh:(i + 1) * Dh]
    for i in range(Hkv):
        k_ref[i] = k[:, i * Dh:(i + 1) * Dh]
        v_ref[i] = v[:, i * Dh:(i + 1) * Dh]


def _rope(t, cos, sin, half):
    t1 = t[:, :half]
    t2 = t[:, half:]
    return jnp.concatenate([t1 * cos - t2 * sin, t1 * sin + t2 * cos], axis=1)


def _attn_kernel(q_ref, k_ref, v_ref, o_ref, *, T, Dh, Bq):
    half = Dh // 2
    qb = pl.program_id(1)
    fr = lax.broadcasted_iota(jnp.int32, (1, half), 1).astype(jnp.float32)
    inv_freq = jnp.exp(fr * (-math.log(10000.0) / half))

    def trig(j):
        posj = (lax.broadcasted_iota(jnp.int32, (Bq, 1), 0)
                + j * Bq).astype(jnp.float32)
        ang = posj * inv_freq
        return jnp.cos(ang), jnp.sin(ang)

    cos_q, sin_q = trig(qb)
    q = _rope(q_ref[0], cos_q, sin_q, half) * (1.0 / math.sqrt(Dh))

    def blk(j):
        cos_k, sin_k = trig(j)
        kj = _rope(k_ref[0, pl.ds(j * Bq, Bq), :], cos_k, sin_k, half)
        s = lax.dot_general(q, kj, (((1,), (1,)), ((), ())),
                            preferred_element_type=jnp.float32)
        vj = v_ref[0, pl.ds(j * Bq, Bq), :]
        return s, vj

    def upd(carry, s, vj):
        m, l, acc = carry
        mn = jnp.maximum(m, jnp.max(s, axis=-1, keepdims=True))
        sc = jnp.exp(m - mn)
        p = jnp.exp(s - mn)
        acc = acc * sc + jnp.dot(p, vj, preferred_element_type=jnp.float32)
        l = l * sc + jnp.sum(p, axis=-1, keepdims=True)
        return mn, l, acc

    def body(j, carry):
        s, vj = blk(j)
        return upd(carry, s, vj)

    m0 = jnp.full((Bq, 1), -1e30, jnp.float32)
    l0 = jnp.zeros((Bq, 1), jnp.float32)
    a0 = jnp.zeros((Bq, Dh), jnp.float32)
    carry = lax.fori_loop(0, qb, body, (m0, l0, a0))
    s, vj = blk(qb)
    ri = lax.broadcasted_iota(jnp.int32, (Bq, Bq), 0)
    ci = lax.broadcasted_iota(jnp.int32, (Bq, Bq), 1)
    s = jnp.where(ci <= ri, s, -1e30)
    m, l, acc = upd(carry, s, vj)
    o_ref[0] = acc / l


def _post_kernel(y_ref, wo_ref, x_ref, rms2_ref, wr_ref,
                 h1_ref, hn_ref, g_ref, aux_ref, *, E, H):
    y = jnp.concatenate([y_ref[i] for i in range(H)], axis=-1)
    h1 = x_ref[...] + jnp.dot(y, wo_ref[...],
                              preferred_element_type=jnp.float32)
    h1_ref[...] = h1
    ms = jnp.mean(h1 * h1, axis=-1, keepdims=True)
    hn = h1 * lax.rsqrt(ms + 1e-6) * rms2_ref[...]
    T = hn.shape[0]
    hn_ref[0:T, :] = hn
    hn_ref[T:T + 8, :] = jnp.zeros((8, hn.shape[1]), jnp.float32)
    logits = jnp.dot(hn, wr_ref[...], preferred_element_type=jnp.float32)
    lm = jnp.max(logits, axis=-1, keepdims=True)
    pe = jnp.exp(logits - lm)
    gates = pe / jnp.sum(pe, axis=-1, keepdims=True)
    g_ref[...] = gates
    load = jnp.mean(gates, axis=0, keepdims=True)
    aux_ref[...] = jnp.mean((load - 1.0 / E) ** 2, axis=-1, keepdims=True)


def _route_kernel(g_ref, r_ref, ri_ref, wb1_ref, wb2_ref, src_ref, *, T, E,
                  capacity, Cpad):
    g = g_ref[...]
    lane = lax.broadcasted_iota(jnp.int32, (T, E), 1).astype(jnp.float32)
    m1 = jnp.max(g, axis=-1, keepdims=True)
    eq1 = g == m1
    e1 = jnp.min(jnp.where(eq1, lane, float(E)), axis=-1, keepdims=True)
    M1 = lane == e1
    g2 = jnp.where(M1, NEG, g)
    m2 = jnp.max(g2, axis=-1, keepdims=True)
    eq2 = g2 == m2
    e2 = jnp.min(jnp.where(eq2, lane, float(E)), axis=-1, keepdims=True)
    M2 = lane == e2
    a = jnp.where(M1 | M2, 1.0, 0.0)
    ti = lax.broadcasted_iota(jnp.int32, (T, T), 0)
    tj = lax.broadcasted_iota(jnp.int32, (T, T), 1)
    Ls = jnp.where(tj < ti, 1.0, 0.0)
    cb = jnp.dot(Ls, a, preferred_element_type=jnp.float32)
    slot1 = jnp.sum(jnp.where(M1, cb, 0.0), axis=-1, keepdims=True)
    slot2 = jnp.sum(jnp.where(M2, cb, 0.0), axis=-1, keepdims=True)
    w1 = jnp.where(slot1 < capacity, m1, 0.0)
    w2 = jnp.where(slot2 < capacity, m2, 0.0)
    z = jnp.zeros_like(e1)
    r_ref[...] = jnp.concatenate([e1, slot1, w1, e2, slot2, w2, z, z], axis=1)
    R = E * Cpad
    d1 = jnp.minimum(e1 * Cpad + slot1, float(R - 1)).astype(jnp.int32)
    d2 = jnp.minimum(e2 * Cpad + slot2, float(R - 1)).astype(jnp.int32)
    zi = jnp.zeros_like(d1)
    ri_ref[...] = jnp.concatenate([d1, d2, zi, zi, zi, zi, zi, zi], axis=1)
    wb1_ref[...] = jnp.concatenate([w1] * 16, axis=1)
    wb2_ref[...] = jnp.concatenate([w2] * 16, axis=1)
    sc_iota = lax.broadcasted_iota(jnp.int32, (T, Cpad), 1).astype(jnp.float32)
    A1 = jnp.where((slot1 == sc_iota) & (sc_iota < capacity), 1.0, 0.0)
    A2 = jnp.where((slot2 == sc_iota) & (sc_iota < capacity), 1.0, 0.0)
    ti = lax.broadcasted_iota(jnp.int32, (T, 1), 0)
    tq = (ti // 16).astype(jnp.float32)
    tr = (ti % 16).astype(jnp.float32)
    M1f = jnp.where(M1, 1.0, 0.0)
    M2f = jnp.where(M2, 1.0, 0.0)
    dn = (((0,), (0,)), ((), ()))

    def dotT(a, b):
        return lax.dot_general(a, b, dn, preferred_element_type=jnp.float32)

    srcsum = (16.0 * (dotT(A1, M1f * tq) + dotT(A2, M2f * tq))
              + dotT(A1, M1f * tr) + dotT(A2, M2f * tr))
    cnt = dotT(A1, M1f) + dotT(A2, M2f)
    src_ref[...] = (srcsum + (1.0 - cnt) * float(T)).astype(jnp.int32)


def _sc_dispatch_body(src_hbm, hn_hbm, buf_hbm, idx_v, rows_v, sem,
                      *, R, RCH):
    c = lax.axis_index("c")
    s = lax.axis_index("s")
    wid = s * SC_NC + c
    rows_per_tile = R // SC_NW
    base = wid * rows_per_tile
    for cc in range(rows_per_tile // RCH):
        off = pl.multiple_of(base + cc * RCH, 8)
        pltpu.sync_copy(src_hbm.at[pl.ds(off, RCH)], idx_v)
        pltpu.async_copy(hn_hbm.at[idx_v], rows_v, sem).wait()
        pltpu.sync_copy(rows_v, buf_hbm.at[pl.ds(off, RCH)])


def _ffn_kernel(buf_ref, w1_ref, w2_ref, w3_ref, y_ref):
    xb = buf_ref[0]
    a = jnp.dot(xb, w1_ref[0], preferred_element_type=jnp.float32)
    b = jnp.dot(xb, w2_ref[0], preferred_element_type=jnp.float32)
    g = b * a * lax.logistic(a)
    contrib = jnp.dot(g, w3_ref[0], preferred_element_type=jnp.float32)

    @pl.when(pl.program_id(1) == 0)
    def _():
        y_ref[0] = contrib

    @pl.when(pl.program_id(1) != 0)
    def _():
        y_ref[0] = y_ref[0] + contrib


def _sc_gather2_body(d1_hbm, d2_hbm, ybuf_hbm, g1_hbm, g2_hbm,
                     d1_v, d2_v, r1_v, r2_v, sem, *, N, TCH):
    c = lax.axis_index("c")
    s = lax.axis_index("s")
    wid = s * SC_NC + c
    tok_per_tile = N // SC_NW
    tbase = wid * tok_per_tile
    for cc in range(tok_per_tile // TCH):
        tb = pl.multiple_of(tbase + cc * TCH, 8)
        pltpu.sync_copy(d1_hbm.at[pl.ds(tb, TCH)], d1_v)
        pltpu.sync_copy(d2_hbm.at[pl.ds(tb, TCH)], d2_v)
        cp1 = pltpu.async_copy(ybuf_hbm.at[d1_v], r1_v, sem)
        cp2 = pltpu.async_copy(ybuf_hbm.at[d2_v], r2_v, sem)
        cp1.wait()
        cp2.wait()
        pltpu.sync_copy(r1_v, g1_hbm.at[pl.ds(tb, TCH)])
        pltpu.sync_copy(r2_v, g2_hbm.at[pl.ds(tb, TCH)])


def _final_kernel(h1_ref, g1_ref, g2_ref, wb1_ref, wb2_ref, o_ref):
    o_ref[...] = (h1_ref[...]
                  + wb1_ref[:, 0:1] * g1_ref[...]
                  + wb2_ref[:, 0:1] * g2_ref[...])


def kernel(x, rms1_w, Wq, Wk, Wv, Wo, rms2_w, router_w, w1, w2, w3):
    B, T, C = x.shape
    H, Hkv, E = NUM_HEADS, KV_HEADS, NUM_EXPERTS
    Dh = C // H
    Hid = w1.shape[2]
    N = B * T
    capacity = max(1, int(CAP_FACTOR * (N * 2) / E))
    Cpad = ((capacity + 7) // 8) * 8
    Hb = Hid // 4

    xf = x.reshape(N, C)
    f32 = jnp.float32

    q, k, v = pl.pallas_call(
        functools.partial(_qkv_kernel, H=H, Hkv=Hkv, Dh=Dh),
        out_shape=[jax.ShapeDtypeStruct((H, N, Dh), f32),
                   jax.ShapeDtypeStruct((Hkv, N, Dh), f32),
                   jax.ShapeDtypeStruct((Hkv, N, Dh), f32)],
    )(xf, rms1_w.reshape(1, C), Wq, Wk, Wv)

    grp = H // Hkv
    Bq = 256
    y = pl.pallas_call(
        functools.partial(_attn_kernel, T=N, Dh=Dh, Bq=Bq),
        grid=(H, N // Bq),
        in_specs=[
            pl.BlockSpec((1, Bq, Dh), lambda h, j: (h, j, 0)),
            pl.BlockSpec((1, N, Dh), lambda h, j: (h // grp, 0, 0)),
            pl.BlockSpec((1, N, Dh), lambda h, j: (h // grp, 0, 0)),
        ],
        out_specs=pl.BlockSpec((1, Bq, Dh), lambda h, j: (h, j, 0)),
        out_shape=jax.ShapeDtypeStruct((H, N, Dh), f32),
    )(q, k, v)

    h1, hn_ext, gates, aux = pl.pallas_call(
        functools.partial(_post_kernel, E=E, H=H),
        out_shape=[jax.ShapeDtypeStruct((N, C), f32),
                   jax.ShapeDtypeStruct((N + 8, C), f32),
                   jax.ShapeDtypeStruct((N, E), f32),
                   jax.ShapeDtypeStruct((1, 1), f32)],
    )(y, Wo, xf, rms2_w.reshape(1, C), router_w)

    R = E * Cpad
    rinfo, rints, wb1, wb2, src8 = pl.pallas_call(
        functools.partial(_route_kernel, T=N, E=E, capacity=capacity,
                          Cpad=Cpad),
        out_shape=[jax.ShapeDtypeStruct((N, 8), f32),
                   jax.ShapeDtypeStruct((N, 8), jnp.int32),
                   jax.ShapeDtypeStruct((N, 16), f32),
                   jax.ShapeDtypeStruct((N, 16), f32),
                   jax.ShapeDtypeStruct((Cpad, E), jnp.int32)],
    )(gates)

    d1 = rints[:, 0]
    d2 = rints[:, 1]
    src_flat = src8.T.reshape(R)
    RCH = (R // SC_NW) // 4
    TCH = 32
    i32 = jnp.int32
    mesh = plsc.VectorSubcoreMesh(core_axis_name="c", subcore_axis_name="s")

    buf = pl.kernel(
        functools.partial(_sc_dispatch_body, R=R, RCH=RCH),
        out_type=jax.ShapeDtypeStruct((R, C), f32),
        mesh=mesh,
        scratch_types=[
            pltpu.VMEM((RCH,), i32),
            pltpu.VMEM((RCH, C), f32),
            pltpu.SemaphoreType.DMA,
        ],
    )(src_flat, hn_ext)

    ybuf = pl.pallas_call(
        _ffn_kernel,
        grid=(E, Hid // Hb),
        in_specs=[pl.BlockSpec((1, Cpad, C), lambda e, j: (e, 0, 0)),
                  pl.BlockSpec((1, C, Hb), lambda e, j: (e, 0, j)),
                  pl.BlockSpec((1, C, Hb), lambda e, j: (e, 0, j)),
                  pl.BlockSpec((1, Hb, C), lambda e, j: (e, j, 0))],
        out_specs=pl.BlockSpec((1, Cpad, C), lambda e, j: (e, 0, 0)),
        out_shape=jax.ShapeDtypeStruct((E, Cpad, C), f32),
    )(buf.reshape(E, Cpad, C), w1, w2, w3)

    g1, g2 = pl.kernel(
        functools.partial(_sc_gather2_body, N=N, TCH=TCH),
        out_type=[jax.ShapeDtypeStruct((N, C), f32),
                  jax.ShapeDtypeStruct((N, C), f32)],
        mesh=mesh,
        scratch_types=[
            pltpu.VMEM((TCH,), i32), pltpu.VMEM((TCH,), i32),
            pltpu.VMEM((TCH, C), f32), pltpu.VMEM((TCH, C), f32),
            pltpu.SemaphoreType.DMA,
        ],
    )(d1, d2, ybuf.reshape(R, C))

    out = pl.pallas_call(
        _final_kernel,
        out_shape=jax.ShapeDtypeStruct((N, C), f32),
    )(h1, g1, g2, wb1, wb2)

    return out.reshape(B, T, C), aux[0, 0]


# speedup vs baseline: 1.8257x; 1.8257x over previous
import functools
import math

import jax
import jax.numpy as jnp
from jax import lax
from jax.experimental import pallas as pl
from jax.experimental.pallas import tpu as pltpu
from jax.experimental.pallas import tpu_sc as plsc

SC_NC = 2
SC_NS = 16
SC_L = 16
SC_NW = SC_NC * SC_NS

NUM_HEADS = 16
KV_HEADS = 4
NUM_EXPERTS = 8
CAP_FACTOR = 1.25
NEG = -1e30


def _qkv_kernel(x_ref, rms1_ref, wq_ref, wk_ref, wv_ref, q_ref, k_ref, v_ref,
                *, H, Hkv, Dh):
    x = x_ref[...]
    ms = jnp.mean(x * x, axis=-1, keepdims=True)
    h = x * lax.rsqrt(ms + 1e-6) * rms1_ref[...]
    q = jnp.dot(h, wq_ref[...], preferred_element_type=jnp.float32)
    k = jnp.dot(h, wk_ref[...], preferred_element_type=jnp.float32)
    v = jnp.dot(h, wv_ref[...], preferred_element_type=jnp.float32)
    for i in range(H):
        q_ref[i] = q[:, i * Dh:(i + 1) * Dh]
    for i in range(Hkv):
        k_ref[i] = k[:, i * Dh:(i + 1) * Dh]
        v_ref[i] = v[:, i * Dh:(i + 1) * Dh]


def _rope(t, cos, sin, half):
    t1 = t[:, :half]
    t2 = t[:, half:]
    return jnp.concatenate([t1 * cos - t2 * sin, t1 * sin + t2 * cos], axis=1)


def _attn_kernel(q_ref, k_ref, v_ref, o_ref, *, T, Dh):
    half = Dh // 2
    q_in = q_ref[0]
    k_in = k_ref[0]
    v_in = v_ref[0]
    pos = lax.broadcasted_iota(jnp.int32, (T, 1), 0).astype(jnp.float32)
    fr = lax.broadcasted_iota(jnp.int32, (1, half), 1).astype(jnp.float32)
    inv_freq = jnp.exp(fr * (-math.log(10000.0) / half))
    ang = pos * inv_freq
    cos = jnp.cos(ang)
    sin = jnp.sin(ang)
    q = _rope(q_in, cos, sin, half)
    k = _rope(k_in, cos, sin, half)
    s = lax.dot_general(q, k, (((1,), (1,)), ((), ())),
                        preferred_element_type=jnp.float32)
    s = s * (1.0 / math.sqrt(Dh))
    ri = lax.broadcasted_iota(jnp.int32, (T, T), 0)
    ci = lax.broadcasted_iota(jnp.int32, (T, T), 1)
    s = jnp.where(ci <= ri, s, jnp.finfo(jnp.float32).min)
    m = jnp.max(s, axis=-1, keepdims=True)
    p = jnp.exp(s - m)
    l = jnp.sum(p, axis=-1, keepdims=True)
    o_ref[0] = jnp.dot(p, v_in, preferred_element_type=jnp.float32) / l


def _post_kernel(y_ref, wo_ref, x_ref, rms2_ref, wr_ref,
                 h1_ref, hn_ref, g_ref, aux_ref, *, E, H):
    y = jnp.concatenate([y_ref[i] for i in range(H)], axis=-1)
    h1 = x_ref[...] + jnp.dot(y, wo_ref[...],
                              preferred_element_type=jnp.float32)
    h1_ref[...] = h1
    ms = jnp.mean(h1 * h1, axis=-1, keepdims=True)
    hn = h1 * lax.rsqrt(ms + 1e-6) * rms2_ref[...]
    T = hn.shape[0]
    hn_ref[0:T, :] = hn
    hn_ref[T:T + 8, :] = jnp.zeros((8, hn.shape[1]), jnp.float32)
    logits = jnp.dot(hn, wr_ref[...], preferred_element_type=jnp.float32)
    lm = jnp.max(logits, axis=-1, keepdims=True)
    pe = jnp.exp(logits - lm)
    gates = pe / jnp.sum(pe, axis=-1, keepdims=True)
    g_ref[...] = gates
    load = jnp.mean(gates, axis=0, keepdims=True)
    aux_ref[...] = jnp.mean((load - 1.0 / E) ** 2, axis=-1, keepdims=True)


def _route_kernel(g_ref, r_ref, ri_ref, wb1_ref, wb2_ref, src_ref, *, T, E,
                  capacity, Cpad):
    g = g_ref[...]
    lane = lax.broadcasted_iota(jnp.int32, (T, E), 1).astype(jnp.float32)
    m1 = jnp.max(g, axis=-1, keepdims=True)
    eq1 = g == m1
    e1 = jnp.min(jnp.where(eq1, lane, float(E)), axis=-1, keepdims=True)
    M1 = lane == e1
    g2 = jnp.where(M1, NEG, g)
    m2 = jnp.max(g2, axis=-1, keepdims=True)
    eq2 = g2 == m2
    e2 = jnp.min(jnp.where(eq2, lane, float(E)), axis=-1, keepdims=True)
    M2 = lane == e2
    a = jnp.where(M1 | M2, 1.0, 0.0)
    ti = lax.broadcasted_iota(jnp.int32, (T, T), 0)
    tj = lax.broadcasted_iota(jnp.int32, (T, T), 1)
    Ls = jnp.where(tj < ti, 1.0, 0.0)
    cb = jnp.dot(Ls, a, preferred_element_type=jnp.float32)
    slot1 = jnp.sum(jnp.where(M1, cb, 0.0), axis=-1, keepdims=True)
    slot2 = jnp.sum(jnp.where(M2, cb, 0.0), axis=-1, keepdims=True)
    w1 = jnp.where(slot1 < capacity, m1, 0.0)
    w2 = jnp.where(slot2 < capacity, m2, 0.0)
    z = jnp.zeros_like(e1)
    r_ref[...] = jnp.concatenate([e1, slot1, w1, e2, slot2, w2, z, z], axis=1)
    R = E * Cpad
    d1 = jnp.minimum(e1 * Cpad + slot1, float(R - 1)).astype(jnp.int32)
    d2 = jnp.minimum(e2 * Cpad + slot2, float(R - 1)).astype(jnp.int32)
    zi = jnp.zeros_like(d1)
    ri_ref[...] = jnp.concatenate([d1, d2, zi, zi, zi, zi, zi, zi], axis=1)
    wb1_ref[...] = jnp.concatenate([w1] * 16, axis=1)
    wb2_ref[...] = jnp.concatenate([w2] * 16, axis=1)
    sc_iota = lax.broadcasted_iota(jnp.int32, (T, Cpad), 1).astype(jnp.float32)
    A1 = jnp.where((slot1 == sc_iota) & (sc_iota < capacity), 1.0, 0.0)
    A2 = jnp.where((slot2 == sc_iota) & (sc_iota < capacity), 1.0, 0.0)
    ti = lax.broadcasted_iota(jnp.int32, (T, 1), 0)
    tq = (ti // 16).astype(jnp.float32)
    tr = (ti % 16).astype(jnp.float32)
    M1f = jnp.where(M1, 1.0, 0.0)
    M2f = jnp.where(M2, 1.0, 0.0)
    dn = (((0,), (0,)), ((), ()))

    def dotT(a, b):
        return lax.dot_general(a, b, dn, preferred_element_type=jnp.float32)

    srcsum = (16.0 * (dotT(A1, M1f * tq) + dotT(A2, M2f * tq))
              + dotT(A1, M1f * tr) + dotT(A2, M2f * tr))
    cnt = dotT(A1, M1f) + dotT(A2, M2f)
    src_ref[...] = (srcsum + (1.0 - cnt) * float(T)).astype(jnp.int32)


def _sc_dispatch_body(src_hbm, hn_hbm, buf_hbm, idx_v, rows_v, sem,
                      *, R, RCH):
    c = lax.axis_index("c")
    s = lax.axis_index("s")
    wid = s * SC_NC + c
    rows_per_tile = R // SC_NW
    base = wid * rows_per_tile
    for cc in range(rows_per_tile // RCH):
        off = pl.multiple_of(base + cc * RCH, 8)
        pltpu.sync_copy(src_hbm.at[pl.ds(off, RCH)], idx_v)
        pltpu.async_copy(hn_hbm.at[idx_v], rows_v, sem).wait()
        pltpu.sync_copy(rows_v, buf_hbm.at[pl.ds(off, RCH)])


def _ffn_kernel(buf_ref, w1_ref, w2_ref, w3_ref, y_ref):
    xb = buf_ref[0]
    a = jnp.dot(xb, w1_ref[0], preferred_element_type=jnp.float32)
    b = jnp.dot(xb, w2_ref[0], preferred_element_type=jnp.float32)
    g = b * a * lax.logistic(a)
    contrib = jnp.dot(g, w3_ref[0], preferred_element_type=jnp.float32)

    @pl.when(pl.program_id(1) == 0)
    def _():
        y_ref[0] = contrib

    @pl.when(pl.program_id(1) != 0)
    def _():
        y_ref[0] = y_ref[0] + contrib


def _sc_gather2_body(d1_hbm, d2_hbm, ybuf_hbm, g1_hbm, g2_hbm,
                     d1_v, d2_v, r1_v, r2_v, sem, *, N, TCH):
    c = lax.axis_index("c")
    s = lax.axis_index("s")
    wid = s * SC_NC + c
    tok_per_tile = N // SC_NW
    tbase = wid * tok_per_tile
    for cc in range(tok_per_tile // TCH):
        tb = pl.multiple_of(tbase + cc * TCH, 8)
        pltpu.sync_copy(d1_hbm.at[pl.ds(tb, TCH)], d1_v)
        pltpu.sync_copy(d2_hbm.at[pl.ds(tb, TCH)], d2_v)
        cp1 = pltpu.async_copy(ybuf_hbm.at[d1_v], r1_v, sem)
        cp2 = pltpu.async_copy(ybuf_hbm.at[d2_v], r2_v, sem)
        cp1.wait()
        cp2.wait()
        pltpu.sync_copy(r1_v, g1_hbm.at[pl.ds(tb, TCH)])
        pltpu.sync_copy(r2_v, g2_hbm.at[pl.ds(tb, TCH)])


def _final_kernel(h1_ref, g1_ref, g2_ref, wb1_ref, wb2_ref, o_ref):
    o_ref[...] = (h1_ref[...]
                  + wb1_ref[:, 0:1] * g1_ref[...]
                  + wb2_ref[:, 0:1] * g2_ref[...])


def kernel(x, rms1_w, Wq, Wk, Wv, Wo, rms2_w, router_w, w1, w2, w3):
    B, T, C = x.shape
    H, Hkv, E = NUM_HEADS, KV_HEADS, NUM_EXPERTS
    Dh = C // H
    Hid = w1.shape[2]
    N = B * T
    capacity = max(1, int(CAP_FACTOR * (N * 2) / E))
    Cpad = ((capacity + 7) // 8) * 8
    Hb = Hid // 4

    xf = x.reshape(N, C)
    f32 = jnp.float32

    q, k, v = pl.pallas_call(
        functools.partial(_qkv_kernel, H=H, Hkv=Hkv, Dh=Dh),
        out_shape=[jax.ShapeDtypeStruct((H, N, Dh), f32),
                   jax.ShapeDtypeStruct((Hkv, N, Dh), f32),
                   jax.ShapeDtypeStruct((Hkv, N, Dh), f32)],
    )(xf, rms1_w.reshape(1, C), Wq, Wk, Wv)

    grp = H // Hkv
    y = pl.pallas_call(
        functools.partial(_attn_kernel, T=N, Dh=Dh),
        grid=(H,),
        in_specs=[
            pl.BlockSpec((1, N, Dh), lambda h: (h, 0, 0)),
            pl.BlockSpec((1, N, Dh), lambda h: (h // grp, 0, 0)),
            pl.BlockSpec((1, N, Dh), lambda h: (h // grp, 0, 0)),
        ],
        out_specs=pl.BlockSpec((1, N, Dh), lambda h: (h, 0, 0)),
        out_shape=jax.ShapeDtypeStruct((H, N, Dh), f32),
    )(q, k, v)

    h1, hn_ext, gates, aux = pl.pallas_call(
        functools.partial(_post_kernel, E=E, H=H),
        out_shape=[jax.ShapeDtypeStruct((N, C), f32),
                   jax.ShapeDtypeStruct((N + 8, C), f32),
                   jax.ShapeDtypeStruct((N, E), f32),
                   jax.ShapeDtypeStruct((1, 1), f32)],
    )(y, Wo, xf, rms2_w.reshape(1, C), router_w)

    R = E * Cpad
    rinfo, rints, wb1, wb2, src8 = pl.pallas_call(
        functools.partial(_route_kernel, T=N, E=E, capacity=capacity,
                          Cpad=Cpad),
        out_shape=[jax.ShapeDtypeStruct((N, 8), f32),
                   jax.ShapeDtypeStruct((N, 8), jnp.int32),
                   jax.ShapeDtypeStruct((N, 16), f32),
                   jax.ShapeDtypeStruct((N, 16), f32),
                   jax.ShapeDtypeStruct((Cpad, E), jnp.int32)],
    )(gates)

    d1 = rints[:, 0]
    d2 = rints[:, 1]
    src_flat = src8.T.reshape(R)
    RCH = (R // SC_NW) // 4
    TCH = 32
    i32 = jnp.int32
    mesh = plsc.VectorSubcoreMesh(core_axis_name="c", subcore_axis_name="s")

    buf = pl.kernel(
        functools.partial(_sc_dispatch_body, R=R, RCH=RCH),
        out_type=jax.ShapeDtypeStruct((R, C), f32),
        mesh=mesh,
        scratch_types=[
            pltpu.VMEM((RCH,), i32),
            pltpu.VMEM((RCH, C), f32),
            pltpu.SemaphoreType.DMA,
        ],
    )(src_flat, hn_ext)

    ybuf = pl.pallas_call(
        _ffn_kernel,
        grid=(E, Hid // Hb),
        in_specs=[pl.BlockSpec((1, Cpad, C), lambda e, j: (e, 0, 0)),
                  pl.BlockSpec((1, C, Hb), lambda e, j: (e, 0, j)),
                  pl.BlockSpec((1, C, Hb), lambda e, j: (e, 0, j)),
                  pl.BlockSpec((1, Hb, C), lambda e, j: (e, j, 0))],
        out_specs=pl.BlockSpec((1, Cpad, C), lambda e, j: (e, 0, 0)),
        out_shape=jax.ShapeDtypeStruct((E, Cpad, C), f32),
    )(buf.reshape(E, Cpad, C), w1, w2, w3)

    g1, g2 = pl.kernel(
        functools.partial(_sc_gather2_body, N=N, TCH=TCH),
        out_type=[jax.ShapeDtypeStruct((N, C), f32),
                  jax.ShapeDtypeStruct((N, C), f32)],
        mesh=mesh,
        scratch_types=[
            pltpu.VMEM((TCH,), i32), pltpu.VMEM((TCH,), i32),
            pltpu.VMEM((TCH, C), f32), pltpu.VMEM((TCH, C), f32),
            pltpu.SemaphoreType.DMA,
        ],
    )(d1, d2, ybuf.reshape(R, C))

    out = pl.pallas_call(
        _final_kernel,
        out_shape=jax.ShapeDtypeStruct((N, C), f32),
    )(h1, g1, g2, wb1, wb2)

    return out.reshape(B, T, C), aux[0, 0]
